# Initial kernel scaffold; baseline (speedup 1.0000x reference)
import jax
import jax.numpy as jnp
from jax.experimental import pallas as pl


def kernel(position_ids, w0, w1):
    raise NotImplementedError("write your pallas kernel here")


# speedup vs baseline: 1.7748x; 1.7748x over previous
import functools

import jax
import jax.numpy as jnp
from jax import lax
from jax.experimental import pallas as pl
from jax.experimental.pallas import tpu as pltpu
from jax.experimental.pallas import tpu_sc as plsc

AX0, AX1 = 64, 128
D0, D1 = 256, 768
DH = D0 + D1

NC, NS, L = 2, 16, 16
NW = NC * NS


def _make_sc_kernel(n):
    pw = n // NW
    k = 64
    nchunk = pw // k
    mesh = plsc.VectorSubcoreMesh(core_axis_name="c", subcore_axis_name="s")

    @functools.partial(
        pl.kernel,
        mesh=mesh,
        out_type=jax.ShapeDtypeStruct((n, DH), jnp.float32),
        scratch_types=[
            pltpu.VMEM((pw,), jnp.int32),
            pltpu.VMEM((pw,), jnp.int32),
            pltpu.VMEM((pw,), jnp.int32),
            pltpu.VMEM((k, D0), jnp.float32),
            pltpu.VMEM((k, D1), jnp.float32),
            pltpu.SemaphoreType.DMA,
            pltpu.SemaphoreType.DMA,
        ],
    )
    def kern(pos_hbm, w0_hbm, w1_hbm, out_hbm,
             pos_v, idx0_v, idx1_v, buf0, buf1, gsem, wsem):
        wid = lax.axis_index("s") * NC + lax.axis_index("c")
        base = wid * pw
        pltpu.sync_copy(pos_hbm.at[pl.ds(base, pw)], pos_v)
        for i in range(pw // L):
            p16 = pos_v[pl.ds(i * L, L)]
            idx0_v[pl.ds(i * L, L)] = lax.shift_right_logical(p16, 7)
            idx1_v[pl.ds(i * L, L)] = lax.bitwise_and(p16, 127)
        wr = []
        for c in range(nchunk):
            g0 = pltpu.async_copy(w0_hbm.at[idx0_v.at[pl.ds(c * k, k)]],
                                  buf0, gsem)
            g1 = pltpu.async_copy(w1_hbm.at[idx1_v.at[pl.ds(c * k, k)]],
                                  buf1, gsem)
            g0.wait()
            g1.wait()
            for w in wr:
                w.wait()
            row0 = base + c * k
            wr = [
                pltpu.async_copy(
                    buf0, out_hbm.at[pl.ds(row0, k), pl.ds(0, D0)], wsem),
                pltpu.async_copy(
                    buf1, out_hbm.at[pl.ds(row0, k), pl.ds(D0, D1)], wsem),
            ]
        for w in wr:
            w.wait()

    return kern


def kernel(position_ids, w0, w1):
    b, s = position_ids.shape
    n = b * s
    pos = position_ids.reshape(n).astype(jnp.int32)
    w0f = w0.reshape(AX0, D0)
    w1f = w1.reshape(AX1, D1)
    out = _make_sc_kernel(n)(pos, w0f, w1f)
    return out.reshape(b, s, DH)
